# baseline (device time: 103207 ns/iter reference)
import jax
import jax.numpy as jnp
from jax import lax
from jax.experimental import pallas as pl
from jax.experimental.pallas import tpu as pltpu

N_DEV = 16
M_PER = 256
N_COLS = 2048
N_HALF = N_COLS // 2
N_SUBS = 4
N_SUB = N_HALF // N_SUBS

_RING = [0, 1, 5, 9, 13, 14, 10, 6, 2, 3, 7, 11, 15, 12, 8, 4]
_POS = [0] * N_DEV
for _p, _l in enumerate(_RING):
    _POS[_l] = _p


def _gelu(y):
    c = 0.7978845608028654
    return 0.5 * y * (1.0 + jnp.tanh(c * (y + 0.044715 * y * y * y)))


def kernel(x, w_mat):
    ring = jnp.array(_RING, dtype=jnp.int32)
    pos_of = jnp.array(_POS, dtype=jnp.int32)
    me = lax.axis_index("i").astype(jnp.int32)
    p = pos_of[me]
    prev_id = ring[(p - 1) % N_DEV]
    next_id = ring[(p + 1) % N_DEV]
    hops = jnp.arange(N_DEV - 1, dtype=jnp.int32)
    cw_blocks = ring[(p - 2 - hops) % N_DEV]
    ccw_blocks = ring[(p + 2 + hops) % N_DEV]
    meta = jnp.concatenate([jnp.stack([prev_id, next_id]), cw_blocks, ccw_blocks])

    def body(x_ref, w_ref, meta_ref, out_ref,
             cw_ref, ccw_ref, send_cw, recv_cw, send_ccw, recv_ccw):
        prev_id = meta_ref[0]
        next_id = meta_ref[1]

        barrier = pltpu.get_barrier_semaphore()
        for nbr in (prev_id, next_id):
            pl.semaphore_signal(
                barrier, inc=1, device_id=(nbr,),
                device_id_type=pl.DeviceIdType.MESH,
            )
        pl.semaphore_wait(barrier, 2)

        def partial(block, lo, width):
            return jnp.dot(
                x_ref[pl.ds(block * M_PER, M_PER), :],
                w_ref[:, lo:lo + width],
                preferred_element_type=jnp.float32,
            )

        def make(comm, ssems, rsems, h, j, tgt):
            return pltpu.make_async_remote_copy(
                src_ref=comm.at[h, :, pl.ds(j * N_SUB, N_SUB)],
                dst_ref=comm.at[h + 1, :, pl.ds(j * N_SUB, N_SUB)],
                send_sem=ssems.at[h, j], recv_sem=rsems.at[h, j],
                device_id=(tgt,), device_id_type=pl.DeviceIdType.MESH,
            )

        cw_ref[0] = partial(prev_id, 0, N_HALF).astype(jnp.bfloat16)
        ccw_ref[0] = partial(next_id, N_HALF, N_HALF).astype(jnp.bfloat16)

        started = []

        def launch(comm, ssems, rsems, h, j, tgt):
            r = make(comm, ssems, rsems, h, j, tgt)
            r.start()
            started.append(r)
            return r

        inflight = []
        for j in range(N_SUBS):
            a = launch(cw_ref, send_cw, recv_cw, 0, j, next_id)
            b = launch(ccw_ref, send_ccw, recv_ccw, 0, j, prev_id)
            inflight.append((a, b))

        for s in range(N_DEV - 1):
            last = s == N_DEV - 2
            local_cw = partial(meta_ref[2 + s], 0, N_HALF)
            local_ccw = partial(meta_ref[2 + (N_DEV - 1) + s], N_HALF, N_HALF)
            arriving, inflight = inflight, []

            for j in range(N_SUBS):
                lo = j * N_SUB
                cw_in, ccw_in = arriving[j]
                cw_in.wait_recv()
                acc_cw = (
                    local_cw[:, lo:lo + N_SUB]
                    + cw_ref[s + 1, :, lo:lo + N_SUB].astype(jnp.float32)
                )
                if last:
                    out_ref[:, lo:lo + N_SUB] = _gelu(acc_cw)
                    nxt = None
                else:
                    cw_ref[s + 1, :, lo:lo + N_SUB] = acc_cw.astype(jnp.bfloat16)
                    nxt = launch(cw_ref, send_cw, recv_cw, s + 1, j, next_id)

                ccw_in.wait_recv()
                acc_ccw = (
                    local_ccw[:, lo:lo + N_SUB]
                    + ccw_ref[s + 1, :, lo:lo + N_SUB].astype(jnp.float32)
                )
                if last:
                    out_ref[:, N_HALF + lo:N_HALF + lo + N_SUB] = _gelu(acc_ccw)
                else:
                    ccw_ref[s + 1, :, lo:lo + N_SUB] = acc_ccw.astype(jnp.bfloat16)
                    nxt2 = launch(ccw_ref, send_ccw, recv_ccw, s + 1, j, prev_id)
                    inflight.append((nxt, nxt2))

        for r in started:
            r.wait_send()

    return pl.pallas_call(
        body,
        out_shape=jax.ShapeDtypeStruct((M_PER, N_COLS), jnp.float32),
        in_specs=[
            pl.BlockSpec(memory_space=pltpu.VMEM),
            pl.BlockSpec(memory_space=pltpu.VMEM),
            pl.BlockSpec(memory_space=pltpu.SMEM),
        ],
        out_specs=pl.BlockSpec(memory_space=pltpu.VMEM),
        scratch_shapes=[
            pltpu.VMEM((N_DEV, M_PER, N_HALF), jnp.bfloat16),
            pltpu.VMEM((N_DEV, M_PER, N_HALF), jnp.bfloat16),
            pltpu.SemaphoreType.DMA((N_DEV - 1, N_SUBS)),
            pltpu.SemaphoreType.DMA((N_DEV - 1, N_SUBS)),
            pltpu.SemaphoreType.DMA((N_DEV - 1, N_SUBS)),
            pltpu.SemaphoreType.DMA((N_DEV - 1, N_SUBS)),
        ],
        compiler_params=pltpu.CompilerParams(collective_id=0),
    )(x, w_mat, meta)


# device time: 101988 ns/iter; 1.0120x vs baseline; 1.0120x over previous
import jax
import jax.numpy as jnp
from jax import lax
from jax.experimental import pallas as pl
from jax.experimental.pallas import tpu as pltpu

N_DEV = 16
M_PER = 256
N_COLS = 2048
N_HALF = N_COLS // 2
N_SUB = N_HALF // 2

_RING = [0, 1, 5, 9, 13, 14, 10, 6, 2, 3, 7, 11, 15, 12, 8, 4]
_POS = [0] * N_DEV
for _p, _l in enumerate(_RING):
    _POS[_l] = _p


def _gelu(y):
    c = 0.7978845608028654
    return 0.5 * y * (1.0 + jnp.tanh(c * (y + 0.044715 * y * y * y)))


def kernel(x, w_mat):
    ring = jnp.array(_RING, dtype=jnp.int32)
    pos_of = jnp.array(_POS, dtype=jnp.int32)
    me = lax.axis_index("i").astype(jnp.int32)
    p = pos_of[me]
    prev_id = ring[(p - 1) % N_DEV]
    next_id = ring[(p + 1) % N_DEV]
    hops = jnp.arange(N_DEV - 1, dtype=jnp.int32)
    cw_blocks = ring[(p - 2 - hops) % N_DEV]
    ccw_blocks = ring[(p + 2 + hops) % N_DEV]
    meta = jnp.concatenate([jnp.stack([prev_id, next_id]), cw_blocks, ccw_blocks])

    def body(x_ref, w_ref, meta_ref, out_ref,
             cw_ref, ccw_ref, send_cw, recv_cw, send_ccw, recv_ccw):
        prev_id = meta_ref[0]
        next_id = meta_ref[1]

        barrier = pltpu.get_barrier_semaphore()
        for nbr in (prev_id, next_id):
            pl.semaphore_signal(
                barrier, inc=1, device_id=(nbr,),
                device_id_type=pl.DeviceIdType.MESH,
            )
        pl.semaphore_wait(barrier, 2)

        def partial(block, lo, width, dtype=jnp.bfloat16):
            return jnp.dot(
                x_ref[pl.ds(block * M_PER, M_PER), :],
                w_ref[:, lo:lo + width],
                preferred_element_type=jnp.float32,
            ).astype(dtype)

        def make(comm, ssems, rsems, h, j, tgt):
            return pltpu.make_async_remote_copy(
                src_ref=comm.at[h, :, pl.ds(j * N_SUB, N_SUB)],
                dst_ref=comm.at[h + 1, :, pl.ds(j * N_SUB, N_SUB)],
                send_sem=ssems.at[h, j], recv_sem=rsems.at[h, j],
                device_id=(tgt,), device_id_type=pl.DeviceIdType.MESH,
            )

        started = []

        def launch(comm, ssems, rsems, h, j, tgt):
            r = make(comm, ssems, rsems, h, j, tgt)
            r.start()
            started.append(r)
            return r

        inflight = []
        for j in range(2):
            lo = j * N_SUB
            cw_ref[0, :, lo:lo + N_SUB] = partial(prev_id, lo, N_SUB)
            a = launch(cw_ref, send_cw, recv_cw, 0, j, next_id)
            ccw_ref[0, :, lo:lo + N_SUB] = partial(next_id, N_HALF + lo, N_SUB)
            b = launch(ccw_ref, send_ccw, recv_ccw, 0, j, prev_id)
            inflight.append((a, b))

        for s in range(N_DEV - 1):
            last = s == N_DEV - 2
            acc_dtype = jnp.float32 if last else jnp.bfloat16
            local_cw = partial(meta_ref[2 + s], 0, N_HALF, acc_dtype)
            local_ccw = partial(meta_ref[2 + (N_DEV - 1) + s], N_HALF, N_HALF, acc_dtype)
            arriving, inflight = inflight, []

            for j in range(2):
                lo = j * N_SUB
                cw_in, ccw_in = arriving[j]
                cw_in.wait_recv()
                if last:
                    out_ref[:, lo:lo + N_SUB] = _gelu(
                        local_cw[:, lo:lo + N_SUB]
                        + cw_ref[s + 1, :, lo:lo + N_SUB].astype(jnp.float32)
                    )
                    nxt = None
                else:
                    cw_ref[s + 1, :, lo:lo + N_SUB] = (
                        local_cw[:, lo:lo + N_SUB]
                        + cw_ref[s + 1, :, lo:lo + N_SUB]
                    )
                    nxt = launch(cw_ref, send_cw, recv_cw, s + 1, j, next_id)

                ccw_in.wait_recv()
                if last:
                    out_ref[:, N_HALF + lo:N_HALF + lo + N_SUB] = _gelu(
                        local_ccw[:, lo:lo + N_SUB]
                        + ccw_ref[s + 1, :, lo:lo + N_SUB].astype(jnp.float32)
                    )
                else:
                    ccw_ref[s + 1, :, lo:lo + N_SUB] = (
                        local_ccw[:, lo:lo + N_SUB]
                        + ccw_ref[s + 1, :, lo:lo + N_SUB]
                    )
                    nxt2 = launch(ccw_ref, send_ccw, recv_ccw, s + 1, j, prev_id)
                    inflight.append((nxt, nxt2))

        for r in started:
            r.wait_send()

    return pl.pallas_call(
        body,
        out_shape=jax.ShapeDtypeStruct((M_PER, N_COLS), jnp.float32),
        in_specs=[
            pl.BlockSpec(memory_space=pltpu.VMEM),
            pl.BlockSpec(memory_space=pltpu.VMEM),
            pl.BlockSpec(memory_space=pltpu.SMEM),
        ],
        out_specs=pl.BlockSpec(memory_space=pltpu.VMEM),
        scratch_shapes=[
            pltpu.VMEM((N_DEV, M_PER, N_HALF), jnp.bfloat16),
            pltpu.VMEM((N_DEV, M_PER, N_HALF), jnp.bfloat16),
            pltpu.SemaphoreType.DMA((N_DEV - 1, 2)),
            pltpu.SemaphoreType.DMA((N_DEV - 1, 2)),
            pltpu.SemaphoreType.DMA((N_DEV - 1, 2)),
            pltpu.SemaphoreType.DMA((N_DEV - 1, 2)),
        ],
        compiler_params=pltpu.CompilerParams(collective_id=0),
    )(x, w_mat, meta)


# device time: 95895 ns/iter; 1.0763x vs baseline; 1.0635x over previous
import jax
import jax.numpy as jnp
from jax import lax
from jax.experimental import pallas as pl
from jax.experimental.pallas import tpu as pltpu

N_DEV = 16
M_PER = 256
N_COLS = 2048
N_HALF = N_COLS // 2
N_SUBS = 2
N_SUB = N_HALF // N_SUBS


def _gelu(y):
    c = 0.7978845608028654
    return 0.5 * y * (1.0 + jnp.tanh(c * (y + 0.044715 * y * y * y)))


def _mod16(v):
    return lax.rem(v + 2 * N_DEV, N_DEV)


def _pos_to_id(pp):
    v = jnp.where(
        pp <= 4, 4 * (pp - 1) + 1,
        jnp.where(
            pp <= 8, 4 * (8 - pp) + 2,
            jnp.where(pp <= 12, 4 * (pp - 9) + 3, 4 * (16 - pp)),
        ),
    )
    return jnp.where(pp == 0, 0, v).astype(jnp.int32)


def _id_to_pos(me):
    q = lax.rem(me, 4)
    z = lax.div(me, 4)
    return jnp.where(
        q == 0, jnp.where(z == 0, 0, 16 - z),
        jnp.where(q == 1, z + 1, jnp.where(q == 2, 8 - z, z + 9)),
    ).astype(jnp.int32)


def kernel(x, w_mat):
    def body(x_ref, w_ref, out_ref,
             cw_ref, ccw_ref, send_cw, recv_cw, send_ccw, recv_ccw):
        me = lax.axis_index("i").astype(jnp.int32)
        p = _id_to_pos(me)
        prev_id = _pos_to_id(_mod16(p - 1))
        next_id = _pos_to_id(_mod16(p + 1))

        barrier = pltpu.get_barrier_semaphore()
        for nbr in (prev_id, next_id):
            pl.semaphore_signal(
                barrier, inc=1, device_id=(nbr,),
                device_id_type=pl.DeviceIdType.MESH,
            )
        pl.semaphore_wait(barrier, 2)

        def partial(block, lo, width, dtype=jnp.bfloat16):
            return jnp.dot(
                x_ref[pl.ds(block * M_PER, M_PER), :],
                w_ref[:, lo:lo + width],
                preferred_element_type=jnp.float32,
            ).astype(dtype)

        def make(comm, ssems, rsems, h, j, tgt):
            return pltpu.make_async_remote_copy(
                src_ref=comm.at[h, :, pl.ds(j * N_SUB, N_SUB)],
                dst_ref=comm.at[h + 1, :, pl.ds(j * N_SUB, N_SUB)],
                send_sem=ssems.at[h, j], recv_sem=rsems.at[h, j],
                device_id=(tgt,), device_id_type=pl.DeviceIdType.MESH,
            )

        started = []

        def launch(comm, ssems, rsems, h, j, tgt):
            r = make(comm, ssems, rsems, h, j, tgt)
            r.start()
            started.append(r)
            return r

        inflight = []
        for j in range(N_SUBS):
            lo = j * N_SUB
            cw_ref[0, :, lo:lo + N_SUB] = partial(prev_id, lo, N_SUB)
            a = launch(cw_ref, send_cw, recv_cw, 0, j, next_id)
            ccw_ref[0, :, lo:lo + N_SUB] = partial(next_id, N_HALF + lo, N_SUB)
            b = launch(ccw_ref, send_ccw, recv_ccw, 0, j, prev_id)
            inflight.append((a, b))

        for s in range(N_DEV - 1):
            last = s == N_DEV - 2
            b_cw = _pos_to_id(_mod16(p - 2 - s))
            b_ccw = _pos_to_id(_mod16(p + 2 + s))
            acc_dtype = jnp.float32 if last else jnp.bfloat16
            local_cw = partial(b_cw, 0, N_HALF, acc_dtype)
            local_ccw = partial(b_ccw, N_HALF, N_HALF, acc_dtype)
            arriving, inflight = inflight, []

            for j in range(N_SUBS):
                lo = j * N_SUB
                cw_in, ccw_in = arriving[j]
                cw_in.wait_recv()
                if last:
                    out_ref[:, lo:lo + N_SUB] = _gelu(
                        local_cw[:, lo:lo + N_SUB]
                        + cw_ref[s + 1, :, lo:lo + N_SUB].astype(jnp.float32)
                    )
                    nxt = None
                else:
                    cw_ref[s + 1, :, lo:lo + N_SUB] = (
                        local_cw[:, lo:lo + N_SUB]
                        + cw_ref[s + 1, :, lo:lo + N_SUB]
                    )
                    nxt = launch(cw_ref, send_cw, recv_cw, s + 1, j, next_id)

                ccw_in.wait_recv()
                if last:
                    out_ref[:, N_HALF + lo:N_HALF + lo + N_SUB] = _gelu(
                        local_ccw[:, lo:lo + N_SUB]
                        + ccw_ref[s + 1, :, lo:lo + N_SUB].astype(jnp.float32)
                    )
                else:
                    ccw_ref[s + 1, :, lo:lo + N_SUB] = (
                        local_ccw[:, lo:lo + N_SUB]
                        + ccw_ref[s + 1, :, lo:lo + N_SUB]
                    )
                    nxt2 = launch(ccw_ref, send_ccw, recv_ccw, s + 1, j, prev_id)
                    inflight.append((nxt, nxt2))

        for r in started:
            r.wait_send()

    return pl.pallas_call(
        body,
        out_shape=jax.ShapeDtypeStruct((M_PER, N_COLS), jnp.float32),
        in_specs=[
            pl.BlockSpec(memory_space=pltpu.VMEM),
            pl.BlockSpec(memory_space=pltpu.VMEM),
        ],
        out_specs=pl.BlockSpec(memory_space=pltpu.VMEM),
        scratch_shapes=[
            pltpu.VMEM((N_DEV, M_PER, N_HALF), jnp.bfloat16),
            pltpu.VMEM((N_DEV, M_PER, N_HALF), jnp.bfloat16),
            pltpu.SemaphoreType.DMA((N_DEV - 1, N_SUBS)),
            pltpu.SemaphoreType.DMA((N_DEV - 1, N_SUBS)),
            pltpu.SemaphoreType.DMA((N_DEV - 1, N_SUBS)),
            pltpu.SemaphoreType.DMA((N_DEV - 1, N_SUBS)),
        ],
        compiler_params=pltpu.CompilerParams(collective_id=0),
    )(x, w_mat)
